# baseline (speedup 1.0000x reference)
import functools

import jax
import jax.numpy as jnp
from jax import lax
from jax.experimental import pallas as pl
from jax.experimental.pallas import tpu as pltpu
from jax.experimental.pallas import tpu_sc as plsc

LANES = 16


@functools.lru_cache(maxsize=None)
def _make_gather_mean(B, L, D, V):
    info = plsc.get_sparse_core_info()
    NC, NS = info.num_cores, info.num_subcores
    NW = NC * NS
    rows_per_w = B // NW
    G = 64
    n_groups = rows_per_w // G
    DW = D // 2
    n_vec = D // (2 * LANES)
    C0 = 96
    C1 = L - C0
    inv_l = 1.0 / float(L)
    himask = jnp.int32(-65536)

    mesh = plsc.VectorSubcoreMesh(core_axis_name="c", subcore_axis_name="s")

    @functools.partial(
        pl.kernel,
        out_type=jax.ShapeDtypeStruct((B, D), jnp.float32),
        mesh=mesh,
        compiler_params=pltpu.CompilerParams(
            needs_layout_passes=False, use_tc_tiling_on_sc=False),
        scratch_types=[
            pltpu.VMEM((2, G, L), jnp.int32),
            pltpu.VMEM((C0, D), jnp.bfloat16),
            pltpu.VMEM((C1, D), jnp.bfloat16),
            pltpu.VMEM((G, D), jnp.float32),
            pltpu.SemaphoreType.DMA,
            pltpu.SemaphoreType.DMA,
            pltpu.SemaphoreType.DMA,
        ],
    )
    def gather_mean(idx_hbm, table_hbm, out_hbm, idx_v, rows0, rows1, out_v,
                    gsem0, gsem1, isem):
        wid = lax.axis_index("s") * NC + lax.axis_index("c")
        base = wid * rows_per_w
        pltpu.sync_copy(idx_hbm.at[pl.ds(base, G)], idx_v.at[0])

        def accum(rows_ref, n, acc):
            def pair_body(t, a):
                new = list(a)
                for j in range(n_vec):
                    xa = rows_ref[2 * t, pl.ds(j * 2 * LANES, 2 * LANES)]
                    xb = rows_ref[2 * t + 1, pl.ds(j * 2 * LANES, 2 * LANES)]
                    w = plsc.bitcast(xa + xb, jnp.int32)
                    lo = plsc.bitcast(lax.shift_left(w, 16), jnp.float32)
                    hi = plsc.bitcast(lax.bitwise_and(w, himask), jnp.float32)
                    new[2 * j] = new[2 * j] + lo
                    new[2 * j + 1] = new[2 * j + 1] + hi
                return tuple(new)
            return plsc.parallel_loop(0, n // 2, carry=acc, unroll=2)(pair_body)

        def group_body(g, carry):
            p = lax.rem(g, 2)
            gbase = base + g * G

            @pl.when(g + 1 < n_groups)
            def _prefetch_idx():
                pltpu.async_copy(
                    idx_hbm.at[pl.ds(gbase + G, G)], idx_v.at[1 - p], isem)

            pltpu.async_copy(
                table_hbm.at[idx_v.at[p, 0, pl.ds(0, C0)]], rows0, gsem0)

            def row_body(r, carry):
                pltpu.async_copy(
                    table_hbm.at[idx_v.at[p, r, pl.ds(C0, C1)]], rows1, gsem1)
                pltpu.make_async_copy(
                    table_hbm.at[idx_v.at[p, r, pl.ds(0, C0)]],
                    rows0, gsem0).wait()
                acc = tuple(jnp.zeros((LANES,), jnp.float32)
                            for _ in range(2 * n_vec))
                acc = accum(rows0, C0, acc)

                @pl.when(r + 1 < G)
                def _issue_next():
                    pltpu.async_copy(
                        table_hbm.at[idx_v.at[p, r + 1, pl.ds(0, C0)]],
                        rows0, gsem0)

                pltpu.make_async_copy(
                    table_hbm.at[idx_v.at[p, r, pl.ds(C0, C1)]],
                    rows1, gsem1).wait()
                acc = accum(rows1, C1, acc)
                for j in range(n_vec):
                    out_v[r, pl.ds(j * LANES, LANES)] = acc[2 * j] * inv_l
                    out_v[r, pl.ds(DW + j * LANES, LANES)] = \
                        acc[2 * j + 1] * inv_l
                return carry

            lax.fori_loop(0, G, row_body, 0)
            pltpu.sync_copy(out_v, out_hbm.at[pl.ds(gbase, G)])

            @pl.when(g + 1 < n_groups)
            def _wait_idx():
                pltpu.make_async_copy(
                    idx_hbm.at[pl.ds(gbase + G, G)], idx_v.at[1 - p], isem).wait()

            return carry

        lax.fori_loop(0, n_groups, group_body, 0)

    return gather_mean


def kernel(index_tensor_list, table):
    B, L = index_tensor_list.shape
    D = 128
    V = table.shape[0]
    idx = index_tensor_list
    if idx.dtype != jnp.int32:
        idx = idx.astype(jnp.int32)
    table_bf = table[:, :D].astype(jnp.bfloat16)
    table_i = jnp.stack(
        [table_bf[:, :D // 2], table_bf[:, D // 2:]], axis=-1).reshape(V, D)
    fn = _make_gather_mean(B, L, D, V)
    return fn(idx, table_i)


# speedup vs baseline: 1.1305x; 1.1305x over previous
import functools

import jax
import jax.numpy as jnp
from jax import lax
from jax.experimental import pallas as pl
from jax.experimental.pallas import tpu as pltpu
from jax.experimental.pallas import tpu_sc as plsc

LANES = 16


@functools.lru_cache(maxsize=None)
def _make_gather_mean(B, L, D, V):
    info = plsc.get_sparse_core_info()
    NC, NS = info.num_cores, info.num_subcores
    NW = NC * NS
    rows_per_w = B // NW
    G = 64
    n_groups = rows_per_w // G
    DW = D // 2
    n_vec = DW // LANES
    C0 = 96
    C1 = L - C0
    inv_l = 1.0 / float(L)

    mesh = plsc.VectorSubcoreMesh(core_axis_name="c", subcore_axis_name="s")

    @functools.partial(
        pl.kernel,
        out_type=jax.ShapeDtypeStruct((B, D), jnp.float32),
        mesh=mesh,
        compiler_params=pltpu.CompilerParams(
            needs_layout_passes=False, use_tc_tiling_on_sc=False),
        scratch_types=[
            pltpu.VMEM((2, G, L), jnp.int32),
            pltpu.VMEM((C0, DW), jnp.int32),
            pltpu.VMEM((C1, DW), jnp.int32),
            pltpu.VMEM((G, D), jnp.float32),
            pltpu.SemaphoreType.DMA,
            pltpu.SemaphoreType.DMA,
            pltpu.SemaphoreType.DMA,
        ],
    )
    def gather_mean(idx_hbm, table_hbm, out_hbm, idx_v, rows0, rows1, out_v,
                    gsem0, gsem1, isem):
        wid = lax.axis_index("s") * NC + lax.axis_index("c")
        base = wid * rows_per_w
        pltpu.sync_copy(idx_hbm.at[pl.ds(base, G)], idx_v.at[0])

        def accum(rows_ref, n, acc):
            def tok_body(t, a):
                new = list(a)
                for j in range(n_vec):
                    x = rows_ref[t, pl.ds(j * LANES, LANES)]
                    lo = plsc.bitcast(lax.shift_left(x, 16), jnp.float32)
                    hi = plsc.bitcast(x, jnp.float32)
                    new[2 * j] = new[2 * j] + lo
                    new[2 * j + 1] = new[2 * j + 1] + hi
                return tuple(new)
            return plsc.parallel_loop(0, n, carry=acc, unroll=4)(tok_body)

        def group_body(g, carry):
            p = lax.rem(g, 2)
            gbase = base + g * G

            @pl.when(g + 1 < n_groups)
            def _prefetch_idx():
                pltpu.async_copy(
                    idx_hbm.at[pl.ds(gbase + G, G)], idx_v.at[1 - p], isem)

            pltpu.async_copy(
                table_hbm.at[idx_v.at[p, 0, pl.ds(0, C0)]], rows0, gsem0)

            def row_body(r, carry):
                pltpu.async_copy(
                    table_hbm.at[idx_v.at[p, r, pl.ds(C0, C1)]], rows1, gsem1)
                pltpu.make_async_copy(
                    table_hbm.at[idx_v.at[p, r, pl.ds(0, C0)]],
                    rows0, gsem0).wait()
                acc = tuple(jnp.zeros((LANES,), jnp.float32)
                            for _ in range(2 * n_vec))
                acc = accum(rows0, C0, acc)

                @pl.when(r + 1 < G)
                def _issue_next():
                    pltpu.async_copy(
                        table_hbm.at[idx_v.at[p, r + 1, pl.ds(0, C0)]],
                        rows0, gsem0)

                pltpu.make_async_copy(
                    table_hbm.at[idx_v.at[p, r, pl.ds(C0, C1)]],
                    rows1, gsem1).wait()
                acc = accum(rows1, C1, acc)
                for j in range(n_vec):
                    out_v[r, pl.ds(j * LANES, LANES)] = acc[2 * j] * inv_l
                    out_v[r, pl.ds(DW + j * LANES, LANES)] = \
                        acc[2 * j + 1] * inv_l
                return carry

            lax.fori_loop(0, G, row_body, 0)
            pltpu.sync_copy(out_v, out_hbm.at[pl.ds(gbase, G)])

            @pl.when(g + 1 < n_groups)
            def _wait_idx():
                pltpu.make_async_copy(
                    idx_hbm.at[pl.ds(gbase + G, G)], idx_v.at[1 - p], isem).wait()

            return carry

        lax.fori_loop(0, n_groups, group_body, 0)

    return gather_mean


def kernel(index_tensor_list, table):
    B, L = index_tensor_list.shape
    D = 128
    V = table.shape[0]
    idx = index_tensor_list
    if idx.dtype != jnp.int32:
        idx = idx.astype(jnp.int32)
    table_bf = table[:, :D].astype(jnp.bfloat16)
    table_w = lax.bitcast_convert_type(
        jnp.stack([table_bf[:, :D // 2], table_bf[:, D // 2:]], axis=-1),
        jnp.int32)
    fn = _make_gather_mean(B, L, D, V)
    return fn(idx, table_w)


# speedup vs baseline: 1.4078x; 1.2453x over previous
import functools

import jax
import jax.numpy as jnp
from jax import lax
from jax.experimental import pallas as pl
from jax.experimental.pallas import tpu as pltpu
from jax.experimental.pallas import tpu_sc as plsc

LANES = 16


@functools.lru_cache(maxsize=None)
def _make_gather_mean(B, L, D, V):
    info = plsc.get_sparse_core_info()
    NC, NS = info.num_cores, info.num_subcores
    NW = NC * NS
    rows_per_w = B // NW
    G = 64
    n_groups = rows_per_w // G
    DW = D // 2
    n_vec = DW // LANES
    inv_l = 1.0 / float(L)

    mesh = plsc.VectorSubcoreMesh(core_axis_name="c", subcore_axis_name="s")

    @functools.partial(
        pl.kernel,
        out_type=jax.ShapeDtypeStruct((B, D), jnp.float32),
        mesh=mesh,
        compiler_params=pltpu.CompilerParams(
            needs_layout_passes=False, use_tc_tiling_on_sc=False),
        scratch_types=[
            pltpu.VMEM((2, G, L), jnp.int32),
            pltpu.VMEM((L, DW), jnp.int32),
            pltpu.VMEM((L, DW), jnp.int32),
            pltpu.VMEM((G, D), jnp.float32),
            pltpu.SemaphoreType.DMA,
            pltpu.SemaphoreType.DMA,
            pltpu.SemaphoreType.DMA,
        ],
    )
    def gather_mean(idx_hbm, table_hbm, out_hbm, idx_v, rows0, rows1, out_v,
                    gsem0, gsem1, isem):
        wid = lax.axis_index("s") * NC + lax.axis_index("c")
        base = wid * rows_per_w
        pltpu.sync_copy(idx_hbm.at[pl.ds(base, G)], idx_v.at[0])

        def accum(rows_ref, n, acc):
            def tok_body(t, a):
                new = list(a)
                for j in range(n_vec):
                    x = rows_ref[t, pl.ds(j * LANES, LANES)]
                    lo = plsc.bitcast(lax.shift_left(x, 16), jnp.float32)
                    hi = plsc.bitcast(x, jnp.float32)
                    new[2 * j] = new[2 * j] + lo
                    new[2 * j + 1] = new[2 * j + 1] + hi
                return tuple(new)
            return plsc.parallel_loop(0, n, carry=acc, unroll=4)(tok_body)

        def group_body(g, carry):
            p = lax.rem(g, 2)
            gbase = base + g * G

            @pl.when(g + 1 < n_groups)
            def _prefetch_idx():
                pltpu.async_copy(
                    idx_hbm.at[pl.ds(gbase + G, G)], idx_v.at[1 - p], isem)

            pltpu.async_copy(table_hbm.at[idx_v.at[p, 0]], rows0, gsem0)

            def row_body(rr, carry):
                r = 2 * rr
                pltpu.async_copy(
                    table_hbm.at[idx_v.at[p, r + 1]], rows1, gsem1)
                pltpu.make_async_copy(
                    table_hbm.at[idx_v.at[p, r]], rows0, gsem0).wait()
                acc = tuple(jnp.zeros((LANES,), jnp.float32)
                            for _ in range(2 * n_vec))
                acc = accum(rows0, L, acc)
                for j in range(n_vec):
                    out_v[r, pl.ds(j * LANES, LANES)] = acc[2 * j] * inv_l
                    out_v[r, pl.ds(DW + j * LANES, LANES)] = \
                        acc[2 * j + 1] * inv_l

                @pl.when(r + 2 < G)
                def _issue_next():
                    pltpu.async_copy(
                        table_hbm.at[idx_v.at[p, r + 2]], rows0, gsem0)

                pltpu.make_async_copy(
                    table_hbm.at[idx_v.at[p, r + 1]], rows1, gsem1).wait()
                acc = tuple(jnp.zeros((LANES,), jnp.float32)
                            for _ in range(2 * n_vec))
                acc = accum(rows1, L, acc)
                r = r + 1
                for j in range(n_vec):
                    out_v[r, pl.ds(j * LANES, LANES)] = acc[2 * j] * inv_l
                    out_v[r, pl.ds(DW + j * LANES, LANES)] = \
                        acc[2 * j + 1] * inv_l
                return carry

            lax.fori_loop(0, G // 2, row_body, 0)
            pltpu.sync_copy(out_v, out_hbm.at[pl.ds(gbase, G)])

            @pl.when(g + 1 < n_groups)
            def _wait_idx():
                pltpu.make_async_copy(
                    idx_hbm.at[pl.ds(gbase + G, G)], idx_v.at[1 - p], isem).wait()

            return carry

        lax.fori_loop(0, n_groups, group_body, 0)

    return gather_mean


def kernel(index_tensor_list, table):
    B, L = index_tensor_list.shape
    D = 128
    V = table.shape[0]
    idx = index_tensor_list
    if idx.dtype != jnp.int32:
        idx = idx.astype(jnp.int32)
    table_bf = table[:, :D].astype(jnp.bfloat16)
    table_w = lax.bitcast_convert_type(
        jnp.stack([table_bf[:, :D // 2], table_bf[:, D // 2:]], axis=-1),
        jnp.int32)
    fn = _make_gather_mean(B, L, D, V)
    return fn(idx, table_w)


# speedup vs baseline: 1.5933x; 1.1317x over previous
import functools

import jax
import jax.numpy as jnp
from jax import lax
from jax.experimental import pallas as pl
from jax.experimental.pallas import tpu as pltpu
from jax.experimental.pallas import tpu_sc as plsc

LANES = 16


@functools.lru_cache(maxsize=None)
def _make_gather_mean(B, L, D, V):
    info = plsc.get_sparse_core_info()
    NC, NS = info.num_cores, info.num_subcores
    NW = NC * NS
    rows_per_w = B // NW
    G = 64
    n_groups = rows_per_w // G
    DW = D // 2
    n_vec = DW // LANES
    RB = 2
    inv_l = 1.0 / float(L)

    mesh = plsc.VectorSubcoreMesh(core_axis_name="c", subcore_axis_name="s")

    @functools.partial(
        pl.kernel,
        out_type=jax.ShapeDtypeStruct((B, D), jnp.float32),
        mesh=mesh,
        compiler_params=pltpu.CompilerParams(
            needs_layout_passes=False, use_tc_tiling_on_sc=False),
        scratch_types=[
            pltpu.VMEM((2, G * L), jnp.int32),
            pltpu.VMEM((RB * L, DW), jnp.int32),
            pltpu.VMEM((RB * L, DW), jnp.int32),
            pltpu.VMEM((G, D), jnp.float32),
            pltpu.SemaphoreType.DMA,
            pltpu.SemaphoreType.DMA,
            pltpu.SemaphoreType.DMA,
        ],
    )
    def gather_mean(idx_hbm, table_hbm, out_hbm, idx_v, rows0, rows1, out_v,
                    gsem0, gsem1, isem):
        wid = lax.axis_index("s") * NC + lax.axis_index("c")
        base = wid * rows_per_w
        pltpu.sync_copy(idx_hbm.at[pl.ds(base * L, G * L)], idx_v.at[0])

        def accum(rows_ref, acc):
            def tok_body(t, a):
                new = list(a)
                for j in range(n_vec):
                    x = rows_ref[t, pl.ds(j * LANES, LANES)]
                    lo = plsc.bitcast(lax.shift_left(x, 16), jnp.float32)
                    hi = plsc.bitcast(x, jnp.float32)
                    new[2 * j] = new[2 * j] + lo
                    new[2 * j + 1] = new[2 * j + 1] + hi
                return tuple(new)
            return plsc.parallel_loop(0, L, carry=acc, unroll=4)(tok_body)

        def reduce_store(rows_buf, row0):
            for q in range(RB):
                acc = tuple(jnp.zeros((LANES,), jnp.float32)
                            for _ in range(2 * n_vec))
                acc = accum(rows_buf.at[pl.ds(q * L, L)], acc)
                for j in range(n_vec):
                    out_v[row0 + q, pl.ds(j * LANES, LANES)] = \
                        acc[2 * j] * inv_l
                    out_v[row0 + q, pl.ds(DW + j * LANES, LANES)] = \
                        acc[2 * j + 1] * inv_l

        def group_body(g, carry):
            p = lax.rem(g, 2)
            gbase = base + g * G

            @pl.when(g + 1 < n_groups)
            def _prefetch_idx():
                pltpu.async_copy(
                    idx_hbm.at[pl.ds((gbase + G) * L, G * L)],
                    idx_v.at[1 - p], isem)

            pltpu.async_copy(
                table_hbm.at[idx_v.at[p, pl.ds(0, RB * L)]], rows0, gsem0)

            def blk_body(k, carry):
                r0 = 2 * RB * k
                pltpu.async_copy(
                    table_hbm.at[idx_v.at[p, pl.ds((r0 + RB) * L, RB * L)]],
                    rows1, gsem1)
                pltpu.make_async_copy(
                    table_hbm.at[idx_v.at[p, pl.ds(r0 * L, RB * L)]],
                    rows0, gsem0).wait()
                reduce_store(rows0, r0)

                @pl.when(r0 + 2 * RB < G)
                def _issue_next():
                    pltpu.async_copy(
                        table_hbm.at[
                            idx_v.at[p, pl.ds((r0 + 2 * RB) * L, RB * L)]],
                        rows0, gsem0)

                pltpu.make_async_copy(
                    table_hbm.at[idx_v.at[p, pl.ds((r0 + RB) * L, RB * L)]],
                    rows1, gsem1).wait()
                reduce_store(rows1, r0 + RB)
                return carry

            lax.fori_loop(0, G // (2 * RB), blk_body, 0)
            pltpu.sync_copy(out_v, out_hbm.at[pl.ds(gbase, G)])

            @pl.when(g + 1 < n_groups)
            def _wait_idx():
                pltpu.make_async_copy(
                    idx_hbm.at[pl.ds((gbase + G) * L, G * L)],
                    idx_v.at[1 - p], isem).wait()

            return carry

        lax.fori_loop(0, n_groups, group_body, 0)

    return gather_mean


def kernel(index_tensor_list, table):
    B, L = index_tensor_list.shape
    D = 128
    V = table.shape[0]
    idx = index_tensor_list
    if idx.dtype != jnp.int32:
        idx = idx.astype(jnp.int32)
    idx = idx.reshape(-1)
    table_bf = table[:, :D].astype(jnp.bfloat16)
    table_w = lax.bitcast_convert_type(
        jnp.stack([table_bf[:, :D // 2], table_bf[:, D // 2:]], axis=-1),
        jnp.int32)
    fn = _make_gather_mean(B, L, D, V)
    return fn(idx, table_w)
